# baseline (speedup 1.0000x reference)
import functools

import jax
import jax.numpy as jnp
from jax import lax
from jax.experimental import pallas as pl
from jax.experimental.pallas import tpu as pltpu
from jax.experimental.pallas import tpu_sc as plsc

B = 16384
D = 64
NC = 2
NS = 16
NW = NC * NS
CHUNK = B // NW
IW = 128
KC = CHUNK // IW

_mesh = plsc.VectorSubcoreMesh(core_axis_name="c", subcore_axis_name="s")


@functools.partial(
    pl.kernel,
    out_type=jax.ShapeDtypeStruct((3 * B, D), jnp.float32),
    mesh=_mesh,
    compiler_params=pltpu.CompilerParams(use_tc_tiling_on_sc=False),
    scratch_types=[
        pltpu.VMEM((KC, IW), jnp.int32),
        pltpu.VMEM((KC, IW), jnp.int32),
        pltpu.VMEM((KC, IW), jnp.int32),
        pltpu.VMEM((KC, IW), jnp.int32),
        pltpu.VMEM((KC, IW), jnp.int32),
        pltpu.VMEM((KC, IW), jnp.int32),
        pltpu.VMEM((KC, IW, D), jnp.float32),
        pltpu.VMEM((KC, IW, D), jnp.float32),
        pltpu.VMEM((KC, IW, D), jnp.float32),
        pltpu.SemaphoreType.DMA,
        pltpu.SemaphoreType.DMA,
        pltpu.SemaphoreType.DMA,
        pltpu.SemaphoreType.DMA,
        pltpu.SemaphoreType.DMA,
        pltpu.SemaphoreType.DMA,
        pltpu.SemaphoreType.DMA,
    ],
)
def _kge_sc(h_idx, r_idx, t_idx, entity, relation, out,
            hidx, ridx, tidx, oh, orr, ot, hrows, rrows, trows,
            sih, sir, sit, sgh, sgr, sgt, sout):
    wid = lax.axis_index("s") * NC + lax.axis_index("c")
    rowbase = wid * KC
    ch = pltpu.async_copy(h_idx.at[pl.ds(rowbase, KC)], hidx, sih)
    cr = pltpu.async_copy(r_idx.at[pl.ds(rowbase, KC)], ridx, sir)
    ct = pltpu.async_copy(t_idx.at[pl.ds(rowbase, KC)], tidx, sit)

    gbase = wid * CHUNK
    lanes = lax.iota(jnp.int32, 16)
    for j in range(CHUNK // 16):
        c, o = divmod(j * 16, IW)
        v = 3 * (gbase + j * 16) + 3 * lanes
        oh[c, pl.ds(o, 16)] = v
        orr[c, pl.ds(o, 16)] = v + 1
        ot[c, pl.ds(o, 16)] = v + 2

    ch.wait()
    gh = [pltpu.async_copy(entity.at[hidx.at[j]], hrows.at[j], sgh)
          for j in range(KC)]
    cr.wait()
    gr = [pltpu.async_copy(relation.at[ridx.at[j]], rrows.at[j], sgr)
          for j in range(KC)]
    ct.wait()
    gt = [pltpu.async_copy(entity.at[tidx.at[j]], trows.at[j], sgt)
          for j in range(KC)]

    outs = []
    for d in gh:
        d.wait()
    outs += [pltpu.async_copy(hrows.at[j], out.at[oh.at[j]], sout)
             for j in range(KC)]
    for d in gr:
        d.wait()
    outs += [pltpu.async_copy(rrows.at[j], out.at[orr.at[j]], sout)
             for j in range(KC)]
    for d in gt:
        d.wait()
    outs += [pltpu.async_copy(trows.at[j], out.at[ot.at[j]], sout)
             for j in range(KC)]
    for d in outs:
        d.wait()


def kernel(h_idx, r_idx, t_idx, entity_emb, relation_emb):
    h2 = h_idx.astype(jnp.int32).reshape(NW * KC, IW)
    r2 = r_idx.astype(jnp.int32).reshape(NW * KC, IW)
    t2 = t_idx.astype(jnp.int32).reshape(NW * KC, IW)
    out = _kge_sc(h2, r2, t2, entity_emb, relation_emb)
    return out.reshape(B, 3, D)


# speedup vs baseline: 1.0706x; 1.0706x over previous
import functools

import jax
import jax.numpy as jnp
from jax import lax
from jax.experimental import pallas as pl
from jax.experimental.pallas import tpu as pltpu
from jax.experimental.pallas import tpu_sc as plsc

B = 16384
NE = 1000000
NRL = 1000
D = 64
DP = 128
NC = 2
NS = 16
NW = NC * NS
CHUNK = B // NW
IW = 128
KC = CHUNK // IW

_mesh = plsc.VectorSubcoreMesh(core_axis_name="c", subcore_axis_name="s")


@functools.partial(
    pl.kernel,
    out_type=jax.ShapeDtypeStruct((3 * B, DP), jnp.float32),
    mesh=_mesh,
    compiler_params=pltpu.CompilerParams(use_tc_tiling_on_sc=False),
    scratch_types=[
        pltpu.VMEM((KC, IW), jnp.int32),
        pltpu.VMEM((KC, IW), jnp.int32),
        pltpu.VMEM((KC, IW), jnp.int32),
        pltpu.VMEM((KC, IW), jnp.int32),
        pltpu.VMEM((KC, IW), jnp.int32),
        pltpu.VMEM((KC, IW), jnp.int32),
        pltpu.VMEM((2, IW, DP), jnp.float32),
        pltpu.SemaphoreType.DMA,
        pltpu.SemaphoreType.DMA,
        pltpu.SemaphoreType.DMA,
        pltpu.SemaphoreType.DMA,
        pltpu.SemaphoreType.DMA,
    ],
)
def _kge_sc(h_idx, r_idx, t_idx, entity, relation, out,
            hidx, ridx, tidx, oh, orr, ot, bufs,
            sidx, sg0, sg1, ss0, ss1):
    wid = lax.axis_index("s") * NC + lax.axis_index("c")
    rowbase = wid * KC
    ch = pltpu.async_copy(h_idx.at[pl.ds(rowbase, KC)], hidx, sidx)
    cr = pltpu.async_copy(r_idx.at[pl.ds(rowbase, KC)], ridx, sidx)
    ct = pltpu.async_copy(t_idx.at[pl.ds(rowbase, KC)], tidx, sidx)

    gbase = wid * CHUNK
    lanes = lax.iota(jnp.int32, 16)
    for j in range(CHUNK // 16):
        c, o = divmod(j * 16, IW)
        v = 3 * (gbase + j * 16) + 3 * lanes
        oh[c, pl.ds(o, 16)] = v
        orr[c, pl.ds(o, 16)] = v + 1
        ot[c, pl.ds(o, 16)] = v + 2
    ch.wait()
    cr.wait()
    ct.wait()

    slots = [(entity, hidx, oh), (relation, ridx, orr), (entity, tidx, ot)]
    gsem = [sg0, sg1]
    ssem = [ss0, ss1]
    steps = [(tbl, idx.at[j], oidx.at[j])
             for (tbl, idx, oidx) in slots for j in range(KC)]
    gathers = [None] * len(steps)
    scatters = [None] * len(steps)
    for k, (tbl, idx, oidx) in enumerate(steps):
        b = k % 2
        if k >= 2:
            scatters[k - 2].wait()
        gathers[k] = pltpu.async_copy(tbl.at[idx], bufs.at[b], gsem[b])
        gathers[k].wait()
        scatters[k] = pltpu.async_copy(bufs.at[b], out.at[oidx], ssem[b])
    scatters[-2].wait()
    scatters[-1].wait()


def kernel(h_idx, r_idx, t_idx, entity_emb, relation_emb):
    h2 = h_idx.astype(jnp.int32).reshape(NW * KC, IW)
    r2 = r_idx.astype(jnp.int32).reshape(NW * KC, IW)
    t2 = t_idx.astype(jnp.int32).reshape(NW * KC, IW)
    ent = jnp.pad(entity_emb, ((0, 0), (0, DP - D)))
    rel = jnp.pad(relation_emb, ((0, 0), (0, DP - D)))
    out = _kge_sc(h2, r2, t2, ent, rel)
    return out[:, :D].reshape(B, 3, D)


# speedup vs baseline: 1.4406x; 1.3455x over previous
import functools

import jax
import jax.numpy as jnp
from jax import lax
from jax.experimental import pallas as pl
from jax.experimental.pallas import tpu as pltpu
from jax.experimental.pallas import tpu_sc as plsc

B = 16384
NE = 1000000
NRL = 1000
D = 64
DP = 128
NC = 2
NS = 16
NW = NC * NS
CHUNK = B // NW
TCOLS = (NE + 127) // 128
SLAB = 245
WTC = 4
NWIN = -(-SLAB // WTC)
RT_TC = (NRL + 127) // 128
LISTN = B * 2 + 16
SINK = 3 * B

_mesh = plsc.VectorSubcoreMesh(core_axis_name="c", subcore_axis_name="s")


@functools.partial(
    pl.kernel,
    out_type=jax.ShapeDtypeStruct((3 * B + 16, DP), jnp.float32),
    mesh=_mesh,
    compiler_params=pltpu.CompilerParams(needs_layout_passes=False),
    scratch_types=[
        pltpu.VMEM((B,), jnp.int32),
        pltpu.VMEM((B,), jnp.int32),
        pltpu.VMEM((CHUNK,), jnp.int32),
        pltpu.VMEM((LISTN,), jnp.int32),
        pltpu.VMEM((16,), jnp.int32),
        pltpu.VMEM((WTC, 8, 8, DP), jnp.float32),
        pltpu.VMEM((DP, DP), jnp.float32),
        pltpu.VMEM((1, DP), jnp.int32),
        pltpu.SemaphoreType.DMA,
        pltpu.SemaphoreType.DMA,
        pltpu.SemaphoreType.DMA,
    ],
)
def _kge_sc(h_idx, r_idx, t_idx, ent_t, rel_t, out,
            hbuf, tbuf, rbuf, lst, swl, win, rowbuf, oidx,
            sidx, swin, sflush):
    wid = lax.axis_index("s") * NC + lax.axis_index("c")
    lo = wid * SLAB * 128
    hi = jnp.minimum(lo + SLAB * 128, NE)
    tc0 = wid * SLAB

    cph = pltpu.async_copy(h_idx, hbuf, sidx)
    cpt = pltpu.async_copy(t_idx, tbuf, sidx)
    cpr = pltpu.async_copy(r_idx.at[pl.ds(wid * CHUNK, CHUNK)], rbuf, sidx)

    lanes = lax.iota(jnp.int32, 16)
    l8 = lanes >> 3
    l7 = lanes & 7
    sinkv = SINK + lanes

    zeros16 = jnp.zeros((16,), jnp.int32)

    def reset_oidx():
        for c in range(8):
            plsc.store_scatter(oidx, [zeros16, 16 * c + lanes], sinkv)

    reset_oidx()

    cph.wait()
    cpt.wait()

    def bucket(idxbuf, slot, off0):
        def body(c, off):
            e = idxbuf[pl.ds(16 * c, 16)]
            orow = 3 * (16 * c + lanes) + slot
            m = (e >= lo) & (e < hi)
            packed = ((e - lo) << 16) | orow
            plsc.store_compressed(lst.at[pl.ds(off, 16)], packed, mask=m)
            return off + plsc.all_reduce_population_count(m)[0]
        return lax.fori_loop(0, B // 16, body, off0)

    nent = bucket(hbuf, 0, jnp.int32(0))
    nent = bucket(tbuf, 2, nent)
    plsc.store_scatter(lst, [nent + lanes], jnp.full((16,), -1, jnp.int32))
    nchunks = (nent + 15) >> 4

    lane0 = lanes == 0

    def extract_entry(pkv, tlocv, rb):
        elv = pkv >> 16
        orowv = pkv & 0xFFFF
        ecolv = elv & 127
        rbv = rb + zeros16
        for c in range(4):
            vals = plsc.load_gather(win, [tlocv, 2 * c + l8, l7, ecolv])
            plsc.store_scatter(rowbuf, [rbv, 16 * c + lanes], vals)
        plsc.store_scatter(oidx, [zeros16, rbv], orowv, mask=lane0)
        return rb + 1

    def maybe_flush(rb):
        def flush(_):
            pltpu.async_copy(rowbuf, out.at[oidx.at[0]], sflush).wait()
            reset_oidx()
            return jnp.int32(0)
        return lax.cond(rb >= DP, flush, lambda _: rb, 0)

    def scan_window(wlo, wsz, tbase, rb0):
        def body(c, rb):
            pk = lst[pl.ds(16 * c, 16)]
            el = pk >> 16
            m = (el >= wlo) & (el < wlo + wsz)
            cnt = plsc.all_reduce_population_count(m)[0]

            def hit(rb):
                plsc.store_compressed(swl.at[pl.ds(0, 16)], pk, mask=m)

                def inner(i, rb):
                    hits = swl[pl.ds(0, 16)]
                    pkv = hits.at[i + zeros16].get(
                        mode="promise_in_bounds")
                    tlocv = tbase + ((pkv >> 16) - wlo) // 128
                    return maybe_flush(extract_entry(pkv, tlocv, rb))
                return lax.fori_loop(0, cnt, inner, rb)
            return lax.cond(cnt > 0, hit, lambda rb: rb, rb)
        return lax.fori_loop(0, nchunks, body, rb0)

    def stage_tc(t, tc, src):
        cps = []
        for bk in range(8):
            cps.append(pltpu.async_copy(
                src.at[pl.ds(8 * bk, 8), pl.ds(tc * 128, 128)],
                win.at[t, bk], swin))
        return cps

    def window(w, rb):
        cps = []
        for t in range(WTC):
            tc = jnp.minimum(tc0 + WTC * w + t, TCOLS - 1)
            cps += stage_tc(t, tc, ent_t)
        for cp in cps:
            cp.wait()
        return scan_window(WTC * 128 * w, WTC * 128, 0, rb)

    rb = lax.fori_loop(0, NWIN, window, jnp.int32(0))

    cpr.wait()

    def r_window(w, width, rb):
        for cp in stage_tc(0, jnp.int32(w), rel_t):
            cp.wait()

        def body(c, rb):
            e = rbuf[pl.ds(16 * c, 16)]
            orow = 3 * (wid * CHUNK + 16 * c + lanes) + 1
            m = (e >= w * 128) & (e < w * 128 + width)
            cnt = plsc.all_reduce_population_count(m)[0]
            packed = ((e - w * 128) << 16) | orow

            def hit(rb):
                plsc.store_compressed(swl.at[pl.ds(0, 16)], packed, mask=m)

                def inner(i, rb):
                    hits = swl[pl.ds(0, 16)]
                    pkv = hits.at[i + zeros16].get(
                        mode="promise_in_bounds")
                    return maybe_flush(extract_entry(pkv, zeros16, rb))
                return lax.fori_loop(0, cnt, inner, rb)
            return lax.cond(cnt > 0, hit, lambda rb: rb, rb)
        return lax.fori_loop(0, CHUNK // 16, body, rb)

    for w in range(RT_TC):
        rb = r_window(w, min(128, NRL - w * 128), rb)

    def final_flush(_):
        pltpu.async_copy(rowbuf, out.at[oidx.at[0]], sflush).wait()
        return 0
    lax.cond(rb > 0, final_flush, lambda _: 0, 0)


def kernel(h_idx, r_idx, t_idx, entity_emb, relation_emb):
    out = _kge_sc(
        h_idx.astype(jnp.int32),
        r_idx.astype(jnp.int32),
        t_idx.astype(jnp.int32),
        entity_emb.T,
        relation_emb.T,
    )
    return out[:3 * B, :D].reshape(B, 3, D)
